# baseline (speedup 1.0000x reference)
import functools

import jax
import jax.numpy as jnp
from jax import lax
from jax.experimental import pallas as pl
from jax.experimental.pallas import tpu as pltpu
from jax.experimental.pallas import tpu_sc as plsc

DIM = 256
N_MLP_LAYERS = 3
N_ITERATIONS = 4

NW = 32
CH = 128
OCH = 64
PAD_ROWS = 8

_SC_MESH = plsc.VectorSubcoreMesh(core_axis_name="c", subcore_axis_name="s")
_SC_PARAMS = pltpu.CompilerParams(needs_layout_passes=False)


def _round_up(x, m):
    return (x + m - 1) // m * m


def _mlp_body(x_ref, w_ref, b_ref, deg_ref, o_ref):
    x = x_ref[...]
    h = jnp.maximum(jnp.dot(x, w_ref[0], preferred_element_type=jnp.float32) + b_ref[0], 0.0)
    h = jnp.maximum(jnp.dot(h, w_ref[1], preferred_element_type=jnp.float32) + b_ref[1], 0.0)
    y = jnp.dot(h, w_ref[2], preferred_element_type=jnp.float32) + b_ref[2]
    deg = deg_ref[...]
    rs = jnp.where(deg > 0, lax.rsqrt(jnp.maximum(deg, 1e-30)), 0.0)
    o_ref[...] = y * rs


def _mlp_scaled(x, W, b, deg, bm):
    n = x.shape[0]
    return pl.pallas_call(
        _mlp_body,
        grid=(n // bm,),
        in_specs=[
            pl.BlockSpec((bm, DIM), lambda i: (i, 0)),
            pl.BlockSpec((N_MLP_LAYERS, DIM, DIM), lambda i: (0, 0, 0)),
            pl.BlockSpec((N_MLP_LAYERS, DIM), lambda i: (0, 0)),
            pl.BlockSpec((bm, 1), lambda i: (i, 0)),
        ],
        out_specs=pl.BlockSpec((bm, DIM), lambda i: (i, 0)),
        out_shape=jax.ShapeDtypeStruct((n, DIM), jnp.float32),
    )(x, W, b, deg)


def _upd2_body(x_ref, a_ref, deg_ref, wa_ref, wb_ref, b_ref, o_ref):
    deg = deg_ref[...]
    a = jnp.where(deg > 0, a_ref[...] * lax.rsqrt(jnp.maximum(deg, 1e-30)), 0.0)
    y = jnp.dot(x_ref[...], wa_ref[...], preferred_element_type=jnp.float32)
    y += jnp.dot(a, wb_ref[...], preferred_element_type=jnp.float32)
    o_ref[...] = y + b_ref[...]


def _upd2(x, a, deg, wa, wb, b, bm):
    n = x.shape[0]
    return pl.pallas_call(
        _upd2_body,
        grid=(n // bm,),
        in_specs=[
            pl.BlockSpec((bm, DIM), lambda i: (i, 0)),
            pl.BlockSpec((bm, DIM), lambda i: (i, 0)),
            pl.BlockSpec((bm, 1), lambda i: (i, 0)),
            pl.BlockSpec((DIM, DIM), lambda i: (0, 0)),
            pl.BlockSpec((DIM, DIM), lambda i: (0, 0)),
            pl.BlockSpec((1, DIM), lambda i: (0, 0)),
        ],
        out_specs=pl.BlockSpec((bm, DIM), lambda i: (i, 0)),
        out_shape=jax.ShapeDtypeStruct((n, DIM), jnp.float32),
    )(x, a, deg, wa, wb, b)


def _upd3_body(x_ref, a_ref, deg_ref, s_ref, w0_ref, w1_ref, w2_ref, b_ref, o_ref):
    deg = deg_ref[...]
    a = jnp.where(deg > 0, a_ref[...] * lax.rsqrt(jnp.maximum(deg, 1e-30)), 0.0)
    y = jnp.dot(x_ref[...], w0_ref[...], preferred_element_type=jnp.float32)
    y += jnp.dot(a, w1_ref[...], preferred_element_type=jnp.float32)
    y += jnp.dot(s_ref[...], w2_ref[...], preferred_element_type=jnp.float32)
    o_ref[...] = y + b_ref[...]


def _upd3(x, a, deg, s, w0, w1, w2, b, bm):
    n = x.shape[0]
    return pl.pallas_call(
        _upd3_body,
        grid=(n // bm,),
        in_specs=[
            pl.BlockSpec((bm, DIM), lambda i: (i, 0)),
            pl.BlockSpec((bm, DIM), lambda i: (i, 0)),
            pl.BlockSpec((bm, 1), lambda i: (i, 0)),
            pl.BlockSpec((bm, DIM), lambda i: (i, 0)),
            pl.BlockSpec((DIM, DIM), lambda i: (0, 0)),
            pl.BlockSpec((DIM, DIM), lambda i: (0, 0)),
            pl.BlockSpec((DIM, DIM), lambda i: (0, 0)),
            pl.BlockSpec((1, DIM), lambda i: (0, 0)),
        ],
        out_specs=pl.BlockSpec((bm, DIM), lambda i: (i, 0)),
        out_shape=jax.ShapeDtypeStruct((n, DIM), jnp.float32),
    )(x, a, deg, s, w0, w1, w2, b)


def _wid():
    return lax.axis_index("s") * 2 + lax.axis_index("c")


def _bounds_pair(bounds_v, w):
    lo = bounds_v[pl.ds(w, 16)][0]
    hi = bounds_v[pl.ds(w + 1, 16)][0]
    return lo, hi


_IOTA16 = functools.partial(lax.broadcasted_iota, jnp.int32, (16,), 0)


def _deg_phase(dst_h, bounds_v, dstb_v, hist_v, out_h, r_tile, w):
    e0, e1 = _bounds_pair(bounds_v, w)
    r0 = w * r_tile
    zero16 = jnp.zeros((16,), jnp.float32)

    def zero_body(j, _):
        hist_v[pl.ds(j * 16, 16)] = zero16
        return 0

    lax.fori_loop(0, r_tile // 16, zero_body, 0)

    p0 = (e0 // 8) * 8
    nch = (e1 - p0 + CH - 1) // CH
    iota = _IOTA16()

    def chunk_body(g, _):
        p = p0 + g * CH
        pltpu.sync_copy(dst_h.at[pl.ds(p, CH)], dstb_v.at[pl.ds(0, CH)])

        ones = jnp.ones((16,), jnp.float32)
        for j in range(CH // 16):
            d16 = dstb_v[pl.ds(j * 16, 16)]
            off = d16 - r0
            pos = p + j * 16 + iota
            valid = jnp.logical_and(pos >= e0, pos < e1)
            plsc.addupdate_scatter(hist_v, [off], ones, mask=valid)
        return 0

    lax.fori_loop(0, nch, chunk_body, 0)
    pltpu.sync_copy(hist_v.at[pl.ds(0, r_tile)], out_h.at[pl.ds(r0, r_tile)])


def _make_deg_kernel(e_pad, r_l, r_c, n_l_out, n_c_out):
    @functools.partial(
        pl.kernel,
        mesh=_SC_MESH,
        compiler_params=_SC_PARAMS,
        out_type=(
            jax.ShapeDtypeStruct((n_l_out,), jnp.float32),
            jax.ShapeDtypeStruct((n_c_out,), jnp.float32),
        ),
        scratch_types=[
            pltpu.VMEM((64,), jnp.int32),
            pltpu.VMEM((64,), jnp.int32),
            pltpu.VMEM((CH + 16,), jnp.int32),
            pltpu.VMEM((max(r_l, r_c),), jnp.float32),
        ],
    )
    def deg_kernel(ldst_h, cdst_h, lbounds_h, cbounds_h, ldeg_h, cdeg_h,
                   lb_v, cb_v, dstb_v, hist_v):
        w = _wid()
        pltpu.sync_copy(lbounds_h, lb_v.at[pl.ds(0, 40)])
        pltpu.sync_copy(cbounds_h, cb_v.at[pl.ds(0, 40)])
        _deg_phase(ldst_h, lb_v, dstb_v, hist_v, ldeg_h, r_l, w)
        _deg_phase(cdst_h, cb_v, dstb_v, hist_v, cdeg_h, r_c, w)

    return deg_kernel


def _make_segsum_kernel(n_src, n_dst, e_pad):
    n_out = n_dst + PAD_ROWS

    @functools.partial(
        pl.kernel,
        mesh=_SC_MESH,
        compiler_params=_SC_PARAMS,
        out_type=jax.ShapeDtypeStruct((n_out, DIM), jnp.float32),
        scratch_types=[
            pltpu.VMEM((64,), jnp.int32),
            pltpu.VMEM((CH,), jnp.int32),
            pltpu.VMEM((CH + 16,), jnp.int32),
            pltpu.VMEM((CH, DIM), jnp.float32),
            pltpu.VMEM((OCH, DIM), jnp.float32),
            pltpu.VMEM((OCH + 16,), jnp.int32),
            pltpu.VMEM((OCH,), jnp.int32),
            pltpu.SemaphoreType.DMA,
            pltpu.SemaphoreType.DMA,
        ],
    )
    def segsum_kernel(table_h, src_h, dst_h, bounds_h, out_h,
                      bounds_v, idx_v, dstb_v, rows_v, stage_v, osm_v, oidx_v,
                      gsem, ssem):
        w = _wid()
        pltpu.sync_copy(bounds_h, bounds_v.at[pl.ds(0, 40)])
        e0, e1 = _bounds_pair(bounds_v, w)
        p0 = (e0 // 8) * 8
        nch = (e1 - p0 + CH - 1) // CH

        iota = _IOTA16()
        dummy = n_dst + (iota & (PAD_ROWS - 1))

        def flush(cnt_n):
            for j in range(OCH // 16):
                v = osm_v[pl.ds(j * 16, 16)]
                pos = j * 16 + iota
                oidx_v[pl.ds(j * 16, 16)] = jnp.where(pos < cnt_n, v, dummy)
            pltpu.async_copy(stage_v, out_h.at[oidx_v], ssem).wait()

        zero16 = jnp.zeros((16,), jnp.float32)
        acc0 = tuple(zero16 for _ in range(16))

        def chunk_body(g, carry):
            cur, cnt, acc = carry
            p = p0 + g * CH
            pltpu.sync_copy(src_h.at[pl.ds(p, CH)], idx_v)
            pltpu.sync_copy(dst_h.at[pl.ds(p, CH)], dstb_v.at[pl.ds(0, CH)])
            pltpu.async_copy(table_h.at[idx_v], rows_v, gsem).wait()
            ilo = jnp.maximum(e0 - p, 0)
            ihi = jnp.minimum(e1 - p, CH)

            def edge_body(i, ec):
                cur, cnt, acc = ec
                d = dstb_v[pl.ds(i, 16)][0]
                newseg = d != cur
                emit = jnp.logical_and(newseg, cur >= 0)

                @pl.when(emit)
                def _():
                    for k in range(16):
                        stage_v[cnt, pl.ds(k * 16, 16)] = acc[k]
                    osm_v[pl.ds(cnt, 16)] = jnp.broadcast_to(cur, (16,))

                cnt = cnt + emit.astype(jnp.int32)

                @pl.when(cnt == OCH)
                def _():
                    flush(jnp.int32(OCH))

                cnt = jnp.where(cnt == OCH, 0, cnt)
                new_acc = tuple(
                    jnp.where(newseg, rows_v[i, pl.ds(k * 16, 16)],
                              acc[k] + rows_v[i, pl.ds(k * 16, 16)])
                    for k in range(16)
                )
                return d, cnt, new_acc

            return lax.fori_loop(ilo, ihi, edge_body, (cur, cnt, acc))

        cur, cnt, acc = lax.fori_loop(
            0, nch, chunk_body, (jnp.int32(-1), jnp.int32(0), acc0))

        @pl.when(cur >= 0)
        def _():
            for k in range(16):
                stage_v[cnt, pl.ds(k * 16, 16)] = acc[k]
            osm_v[pl.ds(cnt, 16)] = jnp.broadcast_to(cur, (16,))

        cnt_final = jnp.where(cur >= 0, cnt + 1, cnt)
        flush(cnt_final)

    return segsum_kernel


def kernel(l_size, c_size, l_edge_index, c_edge_index, l_emb, c_emb,
           l2c_W, l2c_b, c2l_W, c2l_b, cu_W, cu_b, lu_W, lu_b):
    ls = l_emb.shape[0]
    cs = c_emb.shape[0]
    n_edges = l_edge_index.shape[0]

    r_l = _round_up((ls + NW - 1) // NW, 16)
    r_c = _round_up((cs + NW - 1) // NW, 16)
    n_l_deg = NW * r_l
    n_c_deg = NW * r_c
    e_pad = _round_up(n_edges + CH, 8)

    big = jnp.int32(0x3FFFFFFF)
    pad_n = e_pad - n_edges
    pad_src = (jnp.arange(pad_n, dtype=jnp.int32) * 97) % jnp.int32(min(ls, cs))
    pad_dst = jnp.full((pad_n,), big, dtype=jnp.int32)

    perm_c = jnp.argsort(c_edge_index)
    csort_dst = jnp.concatenate([c_edge_index[perm_c].astype(jnp.int32), pad_dst])
    csort_src = jnp.concatenate([l_edge_index[perm_c].astype(jnp.int32), pad_src])
    perm_l = jnp.argsort(l_edge_index)
    lsort_dst = jnp.concatenate([l_edge_index[perm_l].astype(jnp.int32), pad_dst])
    lsort_src = jnp.concatenate([c_edge_index[perm_l].astype(jnp.int32), pad_src])

    def bounds_for(dst_sorted, r_tile):
        b = jnp.searchsorted(dst_sorted, jnp.arange(NW + 1, dtype=jnp.int32) * r_tile)
        return jnp.pad(b.astype(jnp.int32), (0, 40 - NW - 1))

    cbounds = bounds_for(csort_dst, r_c)
    lbounds = bounds_for(lsort_dst, r_l)

    deg_kernel = _make_deg_kernel(e_pad, r_l, r_c, n_l_deg, n_c_deg)
    l_deg, c_deg = deg_kernel(lsort_dst, csort_dst, lbounds, cbounds)
    l_deg = l_deg.reshape(n_l_deg, 1)
    c_deg = c_deg.reshape(n_c_deg, 1)

    seg_c = _make_segsum_kernel(ls, cs, e_pad)
    seg_l = _make_segsum_kernel(cs, ls, e_pad)

    cu_Wa, cu_Wb = cu_W[:DIM], cu_W[DIM:]
    lu_W0, lu_W1, lu_W2 = lu_W[:DIM], lu_W[DIM:2 * DIM], lu_W[2 * DIM:]
    cu_b2 = cu_b.reshape(1, DIM)
    lu_b2 = lu_b.reshape(1, DIM)

    l_embs = [l_emb]
    c_embs = [c_emb]
    for _ in range(N_ITERATIONS):
        l_msg = _mlp_scaled(l_emb, l2c_W, l2c_b, l_deg, bm=1000)
        l2c_aggr = seg_c(l_msg, csort_src, csort_dst, cbounds)

        c_msg = _mlp_scaled(c_emb, c2l_W, c2l_b, c_deg, bm=1000)
        c2l_aggr = seg_l(c_msg, lsort_src, lsort_dst, lbounds)

        l2l_msg = l_emb.reshape(ls // 2, 2, DIM)[:, ::-1, :].reshape(ls, DIM)

        c_emb = _upd2(c_emb, l2c_aggr, c_deg, cu_Wa, cu_Wb, cu_b2, bm=1000)
        c_embs.append(c_emb)
        l_emb = _upd3(l_emb, c2l_aggr, l_deg, l2l_msg,
                      lu_W0, lu_W1, lu_W2, lu_b2, bm=1000)
        l_embs.append(l_emb)
    return (tuple(l_embs), tuple(c_embs))


# speedup vs baseline: 1.3681x; 1.3681x over previous
import functools

import jax
import jax.numpy as jnp
from jax import lax
from jax.experimental import pallas as pl
from jax.experimental.pallas import tpu as pltpu
from jax.experimental.pallas import tpu_sc as plsc

DIM = 256
N_MLP_LAYERS = 3
N_ITERATIONS = 4

NW = 32
CH = 128
OCH = 64
PAD_ROWS = 8

_SC_MESH = plsc.VectorSubcoreMesh(core_axis_name="c", subcore_axis_name="s")
_SC_PARAMS = pltpu.CompilerParams(needs_layout_passes=False)


def _round_up(x, m):
    return (x + m - 1) // m * m


def _mlp_body(x_ref, w_ref, b_ref, deg_ref, o_ref):
    x = x_ref[...]
    h = jnp.maximum(jnp.dot(x, w_ref[0], preferred_element_type=jnp.float32) + b_ref[0], 0.0)
    h = jnp.maximum(jnp.dot(h, w_ref[1], preferred_element_type=jnp.float32) + b_ref[1], 0.0)
    y = jnp.dot(h, w_ref[2], preferred_element_type=jnp.float32) + b_ref[2]
    deg = deg_ref[...]
    rs = jnp.where(deg > 0, lax.rsqrt(jnp.maximum(deg, 1e-30)), 0.0)
    o_ref[...] = y * rs


def _mlp_scaled(x, W, b, deg, bm):
    n = x.shape[0]
    return pl.pallas_call(
        _mlp_body,
        grid=(n // bm,),
        in_specs=[
            pl.BlockSpec((bm, DIM), lambda i: (i, 0)),
            pl.BlockSpec((N_MLP_LAYERS, DIM, DIM), lambda i: (0, 0, 0)),
            pl.BlockSpec((N_MLP_LAYERS, DIM), lambda i: (0, 0)),
            pl.BlockSpec((bm, 1), lambda i: (i, 0)),
        ],
        out_specs=pl.BlockSpec((bm, DIM), lambda i: (i, 0)),
        out_shape=jax.ShapeDtypeStruct((n, DIM), jnp.float32),
    )(x, W, b, deg)


def _upd2_body(x_ref, a_ref, deg_ref, wa_ref, wb_ref, b_ref, o_ref):
    deg = deg_ref[...]
    a = jnp.where(deg > 0, a_ref[...] * lax.rsqrt(jnp.maximum(deg, 1e-30)), 0.0)
    y = jnp.dot(x_ref[...], wa_ref[...], preferred_element_type=jnp.float32)
    y += jnp.dot(a, wb_ref[...], preferred_element_type=jnp.float32)
    o_ref[...] = y + b_ref[...]


def _upd2(x, a, deg, wa, wb, b, bm):
    n = x.shape[0]
    return pl.pallas_call(
        _upd2_body,
        grid=(n // bm,),
        in_specs=[
            pl.BlockSpec((bm, DIM), lambda i: (i, 0)),
            pl.BlockSpec((bm, DIM), lambda i: (i, 0)),
            pl.BlockSpec((bm, 1), lambda i: (i, 0)),
            pl.BlockSpec((DIM, DIM), lambda i: (0, 0)),
            pl.BlockSpec((DIM, DIM), lambda i: (0, 0)),
            pl.BlockSpec((1, DIM), lambda i: (0, 0)),
        ],
        out_specs=pl.BlockSpec((bm, DIM), lambda i: (i, 0)),
        out_shape=jax.ShapeDtypeStruct((n, DIM), jnp.float32),
    )(x, a, deg, wa, wb, b)


def _upd3_body(x_ref, a_ref, deg_ref, s_ref, w0_ref, w1_ref, w2_ref, b_ref, o_ref):
    deg = deg_ref[...]
    a = jnp.where(deg > 0, a_ref[...] * lax.rsqrt(jnp.maximum(deg, 1e-30)), 0.0)
    y = jnp.dot(x_ref[...], w0_ref[...], preferred_element_type=jnp.float32)
    y += jnp.dot(a, w1_ref[...], preferred_element_type=jnp.float32)
    y += jnp.dot(s_ref[...], w2_ref[...], preferred_element_type=jnp.float32)
    o_ref[...] = y + b_ref[...]


def _upd3(x, a, deg, s, w0, w1, w2, b, bm):
    n = x.shape[0]
    return pl.pallas_call(
        _upd3_body,
        grid=(n // bm,),
        in_specs=[
            pl.BlockSpec((bm, DIM), lambda i: (i, 0)),
            pl.BlockSpec((bm, DIM), lambda i: (i, 0)),
            pl.BlockSpec((bm, 1), lambda i: (i, 0)),
            pl.BlockSpec((bm, DIM), lambda i: (i, 0)),
            pl.BlockSpec((DIM, DIM), lambda i: (0, 0)),
            pl.BlockSpec((DIM, DIM), lambda i: (0, 0)),
            pl.BlockSpec((DIM, DIM), lambda i: (0, 0)),
            pl.BlockSpec((1, DIM), lambda i: (0, 0)),
        ],
        out_specs=pl.BlockSpec((bm, DIM), lambda i: (i, 0)),
        out_shape=jax.ShapeDtypeStruct((n, DIM), jnp.float32),
    )(x, a, deg, s, w0, w1, w2, b)


def _wid():
    return lax.axis_index("s") * 2 + lax.axis_index("c")


def _bounds_pair(bounds_v, w):
    lo = bounds_v[pl.ds(w, 16)][0]
    hi = bounds_v[pl.ds(w + 1, 16)][0]
    return lo, hi


_IOTA16 = functools.partial(lax.broadcasted_iota, jnp.int32, (16,), 0)


def _deg_phase(dst_h, bounds_v, dstb_v, hist_v, out_h, r_tile, w):
    e0, e1 = _bounds_pair(bounds_v, w)
    r0 = w * r_tile
    zero16 = jnp.zeros((16,), jnp.float32)

    def zero_body(j, _):
        hist_v[pl.ds(j * 16, 16)] = zero16
        return 0

    lax.fori_loop(0, r_tile // 16, zero_body, 0)

    p0 = (e0 // 8) * 8
    nch = (e1 - p0 + CH - 1) // CH
    iota = _IOTA16()

    def chunk_body(g, _):
        p = p0 + g * CH
        pltpu.sync_copy(dst_h.at[pl.ds(p, CH)], dstb_v.at[pl.ds(0, CH)])

        ones = jnp.ones((16,), jnp.float32)
        for j in range(CH // 16):
            d16 = dstb_v[pl.ds(j * 16, 16)]
            off = d16 - r0
            pos = p + j * 16 + iota
            valid = jnp.logical_and(pos >= e0, pos < e1)
            plsc.addupdate_scatter(hist_v, [off], ones, mask=valid)
        return 0

    lax.fori_loop(0, nch, chunk_body, 0)
    pltpu.sync_copy(hist_v.at[pl.ds(0, r_tile)], out_h.at[pl.ds(r0, r_tile)])


def _make_deg_kernel(e_pad, r_l, r_c, n_l_out, n_c_out):
    @functools.partial(
        pl.kernel,
        mesh=_SC_MESH,
        compiler_params=_SC_PARAMS,
        out_type=(
            jax.ShapeDtypeStruct((n_l_out,), jnp.float32),
            jax.ShapeDtypeStruct((n_c_out,), jnp.float32),
        ),
        scratch_types=[
            pltpu.VMEM((64,), jnp.int32),
            pltpu.VMEM((64,), jnp.int32),
            pltpu.VMEM((CH + 16,), jnp.int32),
            pltpu.VMEM((max(r_l, r_c),), jnp.float32),
        ],
    )
    def deg_kernel(ldst_h, cdst_h, lbounds_h, cbounds_h, ldeg_h, cdeg_h,
                   lb_v, cb_v, dstb_v, hist_v):
        w = _wid()
        pltpu.sync_copy(lbounds_h, lb_v.at[pl.ds(0, 40)])
        pltpu.sync_copy(cbounds_h, cb_v.at[pl.ds(0, 40)])
        _deg_phase(ldst_h, lb_v, dstb_v, hist_v, ldeg_h, r_l, w)
        _deg_phase(cdst_h, cb_v, dstb_v, hist_v, cdeg_h, r_c, w)

    return deg_kernel


def _make_segsum_kernel(n_src, n_dst, e_pad):
    n_out = n_dst + PAD_ROWS

    @functools.partial(
        pl.kernel,
        mesh=_SC_MESH,
        compiler_params=_SC_PARAMS,
        out_type=jax.ShapeDtypeStruct((n_out, DIM), jnp.float32),
        scratch_types=[
            pltpu.VMEM((64,), jnp.int32),
            pltpu.VMEM((CH,), jnp.int32),
            pltpu.VMEM((CH,), jnp.int32),
            pltpu.VMEM((CH + 16,), jnp.int32),
            pltpu.VMEM((CH + 16,), jnp.int32),
            pltpu.VMEM((CH, DIM), jnp.float32),
            pltpu.VMEM((CH, DIM), jnp.float32),
            pltpu.VMEM((OCH, DIM), jnp.float32),
            pltpu.VMEM((OCH + 16,), jnp.int32),
            pltpu.VMEM((OCH,), jnp.int32),
            pltpu.SemaphoreType.DMA,
            pltpu.SemaphoreType.DMA,
            pltpu.SemaphoreType.DMA,
        ],
    )
    def segsum_kernel(table_h, src_h, dst_h, bounds_h, out_h,
                      bounds_v, idx0_v, idx1_v, dstb0_v, dstb1_v,
                      rows0_v, rows1_v, stage_v, osm_v, oidx_v,
                      gsem0, gsem1, ssem):
        w = _wid()
        pltpu.sync_copy(bounds_h, bounds_v.at[pl.ds(0, 40)])
        e0, e1 = _bounds_pair(bounds_v, w)
        p0 = (e0 // 8) * 8
        nch = (e1 - p0 + CH - 1) // CH

        def start_gather(c, idx_r, dstb_r, sem):
            p = p0 + c * CH
            pltpu.sync_copy(src_h.at[pl.ds(p, CH)], idx_r)
            pltpu.sync_copy(dst_h.at[pl.ds(p, CH)], dstb_r.at[pl.ds(0, CH)])

        def start_rows(idx_r, rows_r, sem):
            pltpu.make_async_copy(table_h.at[idx_r], rows_r, sem).start()

        iota = _IOTA16()
        dummy = n_dst + (iota & (PAD_ROWS - 1))

        def flush(cnt_n):
            for j in range(OCH // 16):
                v = osm_v[pl.ds(j * 16, 16)]
                pos = j * 16 + iota
                oidx_v[pl.ds(j * 16, 16)] = jnp.where(pos < cnt_n, v, dummy)
            pltpu.async_copy(stage_v, out_h.at[oidx_v], ssem).wait()

        zero16 = jnp.zeros((16,), jnp.float32)
        acc0 = tuple(zero16 for _ in range(16))

        def process_chunk(g, carry, dstb_v, rows_v):
            cur, cnt, acc = carry
            p = p0 + g * CH
            ilo = jnp.maximum(e0 - p, 0)
            ihi = jnp.minimum(e1 - p, CH)

            def edge_body(i, ec):
                cur, cnt, acc = ec
                d = dstb_v[pl.ds(i, 16)][0]
                newseg = d != cur
                emit = jnp.logical_and(newseg, cur >= 0)
                for k in range(16):
                    stage_v[cnt, pl.ds(k * 16, 16)] = acc[k]
                osm_v[pl.ds(cnt, 16)] = jnp.broadcast_to(cur, (16,))
                cnt = cnt + emit.astype(jnp.int32)

                @pl.when(cnt == OCH)
                def _():
                    flush(jnp.int32(OCH))

                cnt = jnp.where(cnt == OCH, 0, cnt)
                new_acc = tuple(
                    jnp.where(newseg, rows_v[i, pl.ds(k * 16, 16)],
                              acc[k] + rows_v[i, pl.ds(k * 16, 16)])
                    for k in range(16)
                )
                return d, cnt, new_acc

            return lax.fori_loop(ilo, ihi, edge_body, (cur, cnt, acc))

        bufs = ((idx0_v, dstb0_v, rows0_v, gsem0),
                (idx1_v, dstb1_v, rows1_v, gsem1))

        @pl.when(nch > 0)
        def _():
            start_gather(0, idx0_v, dstb0_v, gsem0)
            start_rows(idx0_v, rows0_v, gsem0)

        def step(g, carry, cur_b, nxt_b):
            idx_c, dstb_c, rows_c, sem_c = cur_b
            idx_n, dstb_n, rows_n, sem_n = nxt_b

            def live(carry):
                @pl.when(g + 1 < nch)
                def _():
                    start_gather(g + 1, idx_n, dstb_n, sem_n)
                    start_rows(idx_n, rows_n, sem_n)

                pltpu.make_async_copy(table_h.at[idx_c], rows_c, sem_c).wait()
                return process_chunk(g, carry, dstb_c, rows_c)

            return lax.cond(g < nch, live, lambda c: c, carry)

        def pair_body(q, carry):
            carry = step(2 * q, carry, bufs[0], bufs[1])
            carry = step(2 * q + 1, carry, bufs[1], bufs[0])
            return carry

        npairs = (nch + 1) // 2
        cur, cnt, acc = lax.fori_loop(
            0, npairs, pair_body, (jnp.int32(-1), jnp.int32(0), acc0))

        @pl.when(cur >= 0)
        def _():
            for k in range(16):
                stage_v[cnt, pl.ds(k * 16, 16)] = acc[k]
            osm_v[pl.ds(cnt, 16)] = jnp.broadcast_to(cur, (16,))

        cnt_final = jnp.where(cur >= 0, cnt + 1, cnt)
        flush(cnt_final)

    return segsum_kernel


def kernel(l_size, c_size, l_edge_index, c_edge_index, l_emb, c_emb,
           l2c_W, l2c_b, c2l_W, c2l_b, cu_W, cu_b, lu_W, lu_b):
    ls = l_emb.shape[0]
    cs = c_emb.shape[0]
    n_edges = l_edge_index.shape[0]

    r_l = _round_up((ls + NW - 1) // NW, 16)
    r_c = _round_up((cs + NW - 1) // NW, 16)
    n_l_deg = NW * r_l
    n_c_deg = NW * r_c
    e_pad = _round_up(n_edges + CH, 8)

    big = jnp.int32(0x3FFFFFFF)
    pad_n = e_pad - n_edges
    pad_src = (jnp.arange(pad_n, dtype=jnp.int32) * 97) % jnp.int32(min(ls, cs))
    pad_dst = jnp.full((pad_n,), big, dtype=jnp.int32)

    perm_c = jnp.argsort(c_edge_index)
    csort_dst = jnp.concatenate([c_edge_index[perm_c].astype(jnp.int32), pad_dst])
    csort_src = jnp.concatenate([l_edge_index[perm_c].astype(jnp.int32), pad_src])
    perm_l = jnp.argsort(l_edge_index)
    lsort_dst = jnp.concatenate([l_edge_index[perm_l].astype(jnp.int32), pad_dst])
    lsort_src = jnp.concatenate([c_edge_index[perm_l].astype(jnp.int32), pad_src])

    def bounds_for(dst_sorted, r_tile):
        b = jnp.searchsorted(dst_sorted, jnp.arange(NW + 1, dtype=jnp.int32) * r_tile)
        return jnp.pad(b.astype(jnp.int32), (0, 40 - NW - 1))

    cbounds = bounds_for(csort_dst, r_c)
    lbounds = bounds_for(lsort_dst, r_l)

    deg_kernel = _make_deg_kernel(e_pad, r_l, r_c, n_l_deg, n_c_deg)
    l_deg, c_deg = deg_kernel(lsort_dst, csort_dst, lbounds, cbounds)
    l_deg = l_deg.reshape(n_l_deg, 1)
    c_deg = c_deg.reshape(n_c_deg, 1)

    seg_c = _make_segsum_kernel(ls, cs, e_pad)
    seg_l = _make_segsum_kernel(cs, ls, e_pad)

    cu_Wa, cu_Wb = cu_W[:DIM], cu_W[DIM:]
    lu_W0, lu_W1, lu_W2 = lu_W[:DIM], lu_W[DIM:2 * DIM], lu_W[2 * DIM:]
    cu_b2 = cu_b.reshape(1, DIM)
    lu_b2 = lu_b.reshape(1, DIM)

    l_embs = [l_emb]
    c_embs = [c_emb]
    for _ in range(N_ITERATIONS):
        l_msg = _mlp_scaled(l_emb, l2c_W, l2c_b, l_deg, bm=1000)
        l2c_aggr = seg_c(l_msg, csort_src, csort_dst, cbounds)

        c_msg = _mlp_scaled(c_emb, c2l_W, c2l_b, c_deg, bm=1000)
        c2l_aggr = seg_l(c_msg, lsort_src, lsort_dst, lbounds)

        l2l_msg = l_emb.reshape(ls // 2, 2, DIM)[:, ::-1, :].reshape(ls, DIM)

        c_emb = _upd2(c_emb, l2c_aggr, c_deg, cu_Wa, cu_Wb, cu_b2, bm=1000)
        c_embs.append(c_emb)
        l_emb = _upd3(l_emb, c2l_aggr, l_deg, l2l_msg,
                      lu_W0, lu_W1, lu_W2, lu_b2, bm=1000)
        l_embs.append(l_emb)
    return (tuple(l_embs), tuple(c_embs))


# speedup vs baseline: 1.4358x; 1.0495x over previous
import functools

import jax
import jax.numpy as jnp
from jax import lax
from jax.experimental import pallas as pl
from jax.experimental.pallas import tpu as pltpu
from jax.experimental.pallas import tpu_sc as plsc

DIM = 256
N_MLP_LAYERS = 3
N_ITERATIONS = 4

NW = 32
CH = 128
OCH = 64
PAD_ROWS = 8

_SC_MESH = plsc.VectorSubcoreMesh(core_axis_name="c", subcore_axis_name="s")
_SC_PARAMS = pltpu.CompilerParams(needs_layout_passes=False)


def _round_up(x, m):
    return (x + m - 1) // m * m


def _mlp_body(x_ref, w_ref, b_ref, deg_ref, o_ref):
    x = x_ref[...]
    h = jnp.maximum(jnp.dot(x, w_ref[0], preferred_element_type=jnp.float32) + b_ref[0], 0.0)
    h = jnp.maximum(jnp.dot(h, w_ref[1], preferred_element_type=jnp.float32) + b_ref[1], 0.0)
    y = jnp.dot(h, w_ref[2], preferred_element_type=jnp.float32) + b_ref[2]
    deg = deg_ref[...]
    rs = jnp.where(deg > 0, lax.rsqrt(jnp.maximum(deg, 1e-30)), 0.0)
    o_ref[...] = y * rs


def _mlp_scaled(x, W, b, deg, bm):
    n = x.shape[0]
    return pl.pallas_call(
        _mlp_body,
        grid=(n // bm,),
        in_specs=[
            pl.BlockSpec((bm, DIM), lambda i: (i, 0)),
            pl.BlockSpec((N_MLP_LAYERS, DIM, DIM), lambda i: (0, 0, 0)),
            pl.BlockSpec((N_MLP_LAYERS, DIM), lambda i: (0, 0)),
            pl.BlockSpec((bm, 1), lambda i: (i, 0)),
        ],
        out_specs=pl.BlockSpec((bm, DIM), lambda i: (i, 0)),
        out_shape=jax.ShapeDtypeStruct((n, DIM), jnp.float32),
    )(x, W, b, deg)


def _upd2_body(x_ref, a_ref, deg_ref, wa_ref, wb_ref, b_ref, o_ref):
    deg = deg_ref[...]
    a = jnp.where(deg > 0, a_ref[...] * lax.rsqrt(jnp.maximum(deg, 1e-30)), 0.0)
    y = jnp.dot(x_ref[...], wa_ref[...], preferred_element_type=jnp.float32)
    y += jnp.dot(a, wb_ref[...], preferred_element_type=jnp.float32)
    o_ref[...] = y + b_ref[...]


def _upd2(x, a, deg, wa, wb, b, bm):
    n = x.shape[0]
    return pl.pallas_call(
        _upd2_body,
        grid=(n // bm,),
        in_specs=[
            pl.BlockSpec((bm, DIM), lambda i: (i, 0)),
            pl.BlockSpec((bm, DIM), lambda i: (i, 0)),
            pl.BlockSpec((bm, 1), lambda i: (i, 0)),
            pl.BlockSpec((DIM, DIM), lambda i: (0, 0)),
            pl.BlockSpec((DIM, DIM), lambda i: (0, 0)),
            pl.BlockSpec((1, DIM), lambda i: (0, 0)),
        ],
        out_specs=pl.BlockSpec((bm, DIM), lambda i: (i, 0)),
        out_shape=jax.ShapeDtypeStruct((n, DIM), jnp.float32),
    )(x, a, deg, wa, wb, b)


def _upd3_body(x_ref, a_ref, deg_ref, s_ref, w0_ref, w1_ref, w2_ref, b_ref, o_ref):
    deg = deg_ref[...]
    a = jnp.where(deg > 0, a_ref[...] * lax.rsqrt(jnp.maximum(deg, 1e-30)), 0.0)
    y = jnp.dot(x_ref[...], w0_ref[...], preferred_element_type=jnp.float32)
    y += jnp.dot(a, w1_ref[...], preferred_element_type=jnp.float32)
    y += jnp.dot(s_ref[...], w2_ref[...], preferred_element_type=jnp.float32)
    o_ref[...] = y + b_ref[...]


def _upd3(x, a, deg, s, w0, w1, w2, b, bm):
    n = x.shape[0]
    return pl.pallas_call(
        _upd3_body,
        grid=(n // bm,),
        in_specs=[
            pl.BlockSpec((bm, DIM), lambda i: (i, 0)),
            pl.BlockSpec((bm, DIM), lambda i: (i, 0)),
            pl.BlockSpec((bm, 1), lambda i: (i, 0)),
            pl.BlockSpec((bm, DIM), lambda i: (i, 0)),
            pl.BlockSpec((DIM, DIM), lambda i: (0, 0)),
            pl.BlockSpec((DIM, DIM), lambda i: (0, 0)),
            pl.BlockSpec((DIM, DIM), lambda i: (0, 0)),
            pl.BlockSpec((1, DIM), lambda i: (0, 0)),
        ],
        out_specs=pl.BlockSpec((bm, DIM), lambda i: (i, 0)),
        out_shape=jax.ShapeDtypeStruct((n, DIM), jnp.float32),
    )(x, a, deg, s, w0, w1, w2, b)


def _wid():
    return lax.axis_index("s") * 2 + lax.axis_index("c")


def _bounds_pair(bounds_v, w):
    lo = bounds_v[pl.ds(w, 16)][0]
    hi = bounds_v[pl.ds(w + 1, 16)][0]
    return lo, hi


_IOTA16 = functools.partial(lax.broadcasted_iota, jnp.int32, (16,), 0)


def _deg_phase(dst_h, bounds_v, dstb_v, hist_v, out_h, r_tile, w):
    e0, e1 = _bounds_pair(bounds_v, w)
    r0 = w * r_tile
    zero16 = jnp.zeros((16,), jnp.float32)

    def zero_body(j, _):
        hist_v[pl.ds(j * 16, 16)] = zero16
        return 0

    lax.fori_loop(0, r_tile // 16, zero_body, 0)

    p0 = (e0 // 8) * 8
    nch = (e1 - p0 + CH - 1) // CH
    iota = _IOTA16()

    def chunk_body(g, _):
        p = p0 + g * CH
        pltpu.sync_copy(dst_h.at[pl.ds(p, CH)], dstb_v.at[pl.ds(0, CH)])

        ones = jnp.ones((16,), jnp.float32)
        for j in range(CH // 16):
            d16 = dstb_v[pl.ds(j * 16, 16)]
            off = d16 - r0
            pos = p + j * 16 + iota
            valid = jnp.logical_and(pos >= e0, pos < e1)
            plsc.addupdate_scatter(hist_v, [off], ones, mask=valid)
        return 0

    lax.fori_loop(0, nch, chunk_body, 0)
    pltpu.sync_copy(hist_v.at[pl.ds(0, r_tile)], out_h.at[pl.ds(r0, r_tile)])


def _make_deg_kernel(e_pad, r_l, r_c, n_l_out, n_c_out):
    @functools.partial(
        pl.kernel,
        mesh=_SC_MESH,
        compiler_params=_SC_PARAMS,
        out_type=(
            jax.ShapeDtypeStruct((n_l_out,), jnp.float32),
            jax.ShapeDtypeStruct((n_c_out,), jnp.float32),
        ),
        scratch_types=[
            pltpu.VMEM((64,), jnp.int32),
            pltpu.VMEM((64,), jnp.int32),
            pltpu.VMEM((CH + 16,), jnp.int32),
            pltpu.VMEM((max(r_l, r_c),), jnp.float32),
        ],
    )
    def deg_kernel(ldst_h, cdst_h, lbounds_h, cbounds_h, ldeg_h, cdeg_h,
                   lb_v, cb_v, dstb_v, hist_v):
        w = _wid()
        pltpu.sync_copy(lbounds_h, lb_v.at[pl.ds(0, 40)])
        pltpu.sync_copy(cbounds_h, cb_v.at[pl.ds(0, 40)])
        _deg_phase(ldst_h, lb_v, dstb_v, hist_v, ldeg_h, r_l, w)
        _deg_phase(cdst_h, cb_v, dstb_v, hist_v, cdeg_h, r_c, w)

    return deg_kernel


def _make_segsum_kernel(n_src, n_dst, e_pad):
    n_out = n_dst + PAD_ROWS

    @functools.partial(
        pl.kernel,
        mesh=_SC_MESH,
        compiler_params=_SC_PARAMS,
        out_type=jax.ShapeDtypeStruct((n_out, DIM), jnp.float32),
        scratch_types=[
            pltpu.VMEM((64,), jnp.int32),
            pltpu.VMEM((CH,), jnp.int32),
            pltpu.VMEM((CH,), jnp.int32),
            pltpu.VMEM((CH + 16,), jnp.int32),
            pltpu.VMEM((CH + 16,), jnp.int32),
            pltpu.VMEM((CH, DIM), jnp.float32),
            pltpu.VMEM((CH, DIM), jnp.float32),
            pltpu.VMEM((OCH, DIM), jnp.float32),
            pltpu.VMEM((OCH + 16,), jnp.int32),
            pltpu.VMEM((OCH,), jnp.int32),
            pltpu.SemaphoreType.DMA,
            pltpu.SemaphoreType.DMA,
            pltpu.SemaphoreType.DMA,
        ],
    )
    def segsum_kernel(table_h, src_h, dst_h, bounds_h, out_h,
                      bounds_v, idx0_v, idx1_v, dstb0_v, dstb1_v,
                      rows0_v, rows1_v, stage_v, osm_v, oidx_v,
                      gsem0, gsem1, ssem):
        w = _wid()
        pltpu.sync_copy(bounds_h, bounds_v.at[pl.ds(0, 40)])
        e0, e1 = _bounds_pair(bounds_v, w)
        p0 = (e0 // 8) * 8
        nch = (e1 - p0 + CH - 1) // CH

        def start_gather(c, idx_r, dstb_r, sem):
            p = p0 + c * CH
            pltpu.sync_copy(src_h.at[pl.ds(p, CH)], idx_r)
            pltpu.sync_copy(dst_h.at[pl.ds(p, CH)], dstb_r.at[pl.ds(0, CH)])

        def start_rows(idx_r, rows_r, sem):
            pltpu.make_async_copy(table_h.at[idx_r], rows_r, sem).start()

        iota = _IOTA16()
        dummy = n_dst + (iota & (PAD_ROWS - 1))

        def flush(cnt_n):
            for j in range(OCH // 16):
                v = osm_v[pl.ds(j * 16, 16)]
                pos = j * 16 + iota
                oidx_v[pl.ds(j * 16, 16)] = jnp.where(pos < cnt_n, v, dummy)
            pltpu.async_copy(stage_v, out_h.at[oidx_v], ssem).wait()

        zero16 = jnp.zeros((16,), jnp.float32)
        acc0 = tuple(zero16 for _ in range(16))

        def process_chunk(g, carry, dstb_v, rows_v):
            p = p0 + g * CH
            ilo = jnp.maximum(e0 - p, 0)
            ihi = jnp.minimum(e1 - p, CH)

            def edge_step(i, d, ec):
                cur, cnt, acc = ec
                newseg = d != cur
                emit = jnp.logical_and(newseg, cur >= 0)
                for k in range(16):
                    stage_v[cnt, pl.ds(k * 16, 16)] = acc[k]
                osm_v[pl.ds(cnt, 16)] = jnp.broadcast_to(cur, (16,))
                cnt = cnt + emit.astype(jnp.int32)

                @pl.when(cnt == OCH)
                def _():
                    flush(jnp.int32(OCH))

                cnt = jnp.where(cnt == OCH, 0, cnt)
                new_acc = tuple(
                    jnp.where(newseg, rows_v[i, pl.ds(k * 16, 16)],
                              acc[k] + rows_v[i, pl.ds(k * 16, 16)])
                    for k in range(16)
                )
                return d, cnt, new_acc

            nq = (ihi - ilo) // 4

            def quad_body(qq, ec):
                i = ilo + qq * 4
                d4 = dstb_v[pl.ds(i, 16)]
                for j in range(4):
                    ec = edge_step(i + j, d4[j], ec)
                return ec

            carry = lax.fori_loop(0, nq, quad_body, carry)

            def tail_body(i, ec):
                return edge_step(i, dstb_v[pl.ds(i, 16)][0], ec)

            return lax.fori_loop(ilo + nq * 4, ihi, tail_body, carry)

        bufs = ((idx0_v, dstb0_v, rows0_v, gsem0),
                (idx1_v, dstb1_v, rows1_v, gsem1))

        @pl.when(nch > 0)
        def _():
            start_gather(0, idx0_v, dstb0_v, gsem0)
            start_rows(idx0_v, rows0_v, gsem0)

        def step(g, carry, cur_b, nxt_b):
            idx_c, dstb_c, rows_c, sem_c = cur_b
            idx_n, dstb_n, rows_n, sem_n = nxt_b

            def live(carry):
                @pl.when(g + 1 < nch)
                def _():
                    start_gather(g + 1, idx_n, dstb_n, sem_n)
                    start_rows(idx_n, rows_n, sem_n)

                pltpu.make_async_copy(table_h.at[idx_c], rows_c, sem_c).wait()
                return process_chunk(g, carry, dstb_c, rows_c)

            return lax.cond(g < nch, live, lambda c: c, carry)

        def pair_body(q, carry):
            carry = step(2 * q, carry, bufs[0], bufs[1])
            carry = step(2 * q + 1, carry, bufs[1], bufs[0])
            return carry

        npairs = (nch + 1) // 2
        cur, cnt, acc = lax.fori_loop(
            0, npairs, pair_body, (jnp.int32(-1), jnp.int32(0), acc0))

        @pl.when(cur >= 0)
        def _():
            for k in range(16):
                stage_v[cnt, pl.ds(k * 16, 16)] = acc[k]
            osm_v[pl.ds(cnt, 16)] = jnp.broadcast_to(cur, (16,))

        cnt_final = jnp.where(cur >= 0, cnt + 1, cnt)
        flush(cnt_final)

    return segsum_kernel


def kernel(l_size, c_size, l_edge_index, c_edge_index, l_emb, c_emb,
           l2c_W, l2c_b, c2l_W, c2l_b, cu_W, cu_b, lu_W, lu_b):
    ls = l_emb.shape[0]
    cs = c_emb.shape[0]
    n_edges = l_edge_index.shape[0]

    r_l = _round_up((ls + NW - 1) // NW, 16)
    r_c = _round_up((cs + NW - 1) // NW, 16)
    n_l_deg = NW * r_l
    n_c_deg = NW * r_c
    e_pad = _round_up(n_edges + CH, 8)

    big = jnp.int32(0x3FFFFFFF)
    pad_n = e_pad - n_edges
    pad_src = (jnp.arange(pad_n, dtype=jnp.int32) * 97) % jnp.int32(min(ls, cs))
    pad_dst = jnp.full((pad_n,), big, dtype=jnp.int32)

    cd, csrc = lax.sort([c_edge_index.astype(jnp.int32), l_edge_index.astype(jnp.int32)], num_keys=1)
    csort_dst = jnp.concatenate([cd, pad_dst])
    csort_src = jnp.concatenate([csrc, pad_src])
    ld, lsrc = lax.sort([l_edge_index.astype(jnp.int32), c_edge_index.astype(jnp.int32)], num_keys=1)
    lsort_dst = jnp.concatenate([ld, pad_dst])
    lsort_src = jnp.concatenate([lsrc, pad_src])

    def bounds_for(dst_sorted, r_tile):
        b = jnp.searchsorted(dst_sorted, jnp.arange(NW + 1, dtype=jnp.int32) * r_tile)
        return jnp.pad(b.astype(jnp.int32), (0, 40 - NW - 1))

    cbounds = bounds_for(csort_dst, r_c)
    lbounds = bounds_for(lsort_dst, r_l)

    deg_kernel = _make_deg_kernel(e_pad, r_l, r_c, n_l_deg, n_c_deg)
    l_deg, c_deg = deg_kernel(lsort_dst, csort_dst, lbounds, cbounds)
    l_deg = l_deg.reshape(n_l_deg, 1)
    c_deg = c_deg.reshape(n_c_deg, 1)

    seg_c = _make_segsum_kernel(ls, cs, e_pad)
    seg_l = _make_segsum_kernel(cs, ls, e_pad)

    cu_Wa, cu_Wb = cu_W[:DIM], cu_W[DIM:]
    lu_W0, lu_W1, lu_W2 = lu_W[:DIM], lu_W[DIM:2 * DIM], lu_W[2 * DIM:]
    cu_b2 = cu_b.reshape(1, DIM)
    lu_b2 = lu_b.reshape(1, DIM)

    l_embs = [l_emb]
    c_embs = [c_emb]
    for _ in range(N_ITERATIONS):
        l_msg = _mlp_scaled(l_emb, l2c_W, l2c_b, l_deg, bm=1000)
        l2c_aggr = seg_c(l_msg, csort_src, csort_dst, cbounds)

        c_msg = _mlp_scaled(c_emb, c2l_W, c2l_b, c_deg, bm=1000)
        c2l_aggr = seg_l(c_msg, lsort_src, lsort_dst, lbounds)

        l2l_msg = l_emb.reshape(ls // 2, 2, DIM)[:, ::-1, :].reshape(ls, DIM)

        c_emb = _upd2(c_emb, l2c_aggr, c_deg, cu_Wa, cu_Wb, cu_b2, bm=1000)
        c_embs.append(c_emb)
        l_emb = _upd3(l_emb, c2l_aggr, l_deg, l2l_msg,
                      lu_W0, lu_W1, lu_W2, lu_b2, bm=1000)
        l_embs.append(l_emb)
    return (tuple(l_embs), tuple(c_embs))
